# Initial kernel scaffold; baseline (speedup 1.0000x reference)
import jax
import jax.numpy as jnp
from jax.experimental import pallas as pl


def kernel(frame0, frame2, conv1_0w, conv1_0b, conv1_1w, conv1_1b, conv1_2w, conv1_2b, conv2_0w, conv2_0b, conv2_1w, conv2_1b, conv2_2w, conv2_2b, conv3_0w, conv3_0b, conv3_1w, conv3_1b, conv3_2w, conv3_2b, conv4_0w, conv4_0b, conv4_1w, conv4_1b, conv4_2w, conv4_2b, conv5_0w, conv5_0b, conv5_1w, conv5_1b, conv5_2w, conv5_2b, deconv5_0w, deconv5_0b, deconv5_1w, deconv5_1b, deconv5_2w, deconv5_2b, up5_0w, up5_0b, deconv4_0w, deconv4_0b, deconv4_1w, deconv4_1b, deconv4_2w, deconv4_2b, up4_0w, up4_0b, deconv3_0w, deconv3_0b, deconv3_1w, deconv3_1b, deconv3_2w, deconv3_2b, up3_0w, up3_0b, deconv2_0w, deconv2_0b, deconv2_1w, deconv2_1b, deconv2_2w, deconv2_2b, up2_0w, up2_0b, head_l1_w, head_l1_b, head_l2_w, head_l2_b, head_l3_w, head_l3_b, head_l4_w, head_l4_b):
    raise NotImplementedError("write your pallas kernel here")


# speedup vs baseline: 2.7970x; 2.7970x over previous
import functools

import jax
import jax.numpy as jnp
from jax.experimental import pallas as pl
from jax.experimental.pallas import tpu as pltpu

_KS = 5
_K2 = _KS * _KS
_MEANS = jnp.array([0.4631, 0.4352, 0.3990], jnp.float32)
_VMEM_LIMIT = 58 * 1024 * 1024


def _one_group(pad_ref, w_ref, b_ref, relu, W, Rout, ci_off, ci_len, mode9):
    if mode9:
        xc = jnp.concatenate(
            [pad_ref[dy:dy + Rout, dx:dx + W, ci_off:ci_off + ci_len]
             for dy in range(3) for dx in range(3)], axis=-1)
        acc = jax.lax.dot_general(xc.reshape(Rout * W, 9 * ci_len), w_ref[...],
                                  (((1,), (0,)), ((), ())),
                                  preferred_element_type=jnp.float32)
    else:
        xc = jnp.concatenate(
            [pad_ref[:, dx:dx + W, ci_off:ci_off + ci_len] for dx in range(3)],
            axis=-1)
        acc = None
        for dy in range(3):
            lhs = xc[dy:dy + Rout].reshape(Rout * W, 3 * ci_len)
            t = jax.lax.dot_general(lhs, w_ref[dy], (((1,), (0,)), ((), ())),
                                    preferred_element_type=jnp.float32)
            acc = t if acc is None else acc + t
    y = acc + b_ref[...]
    if relu:
        y = jnp.maximum(y, 0.0)
    return y.reshape(Rout, W, w_ref.shape[-1])


def _conv_step(x, pad_ref, groups, relu, row_lo, row_hi, mask, f32_out=False):
    R, W, Cin = x.shape
    pad_ref[:, 0:1, :] = jnp.zeros((R, 1, Cin), jnp.bfloat16)
    pad_ref[:, W + 1:W + 2, :] = jnp.zeros((R, 1, Cin), jnp.bfloat16)
    pad_ref[:, 1:W + 1, :] = x
    Rout = R - 2
    ys = [_one_group(pad_ref, w, b, relu, W, Rout, co, cl, m9)
          for (w, b, co, cl, m9) in groups]
    y = ys[0] if len(ys) == 1 else jnp.concatenate(ys, axis=-1)
    if mask:
        h = mask
        top = y[0:h]
        bot = y[Rout - h:Rout]
        rt = jax.lax.broadcasted_iota(jnp.int32, top.shape, 0)
        top = jnp.where(rt >= row_lo, top, 0.0)
        rb = jax.lax.broadcasted_iota(jnp.int32, bot.shape, 0) + (Rout - h)
        bot = jnp.where(rb < row_hi, bot, 0.0)
        y = jnp.concatenate([top, y[h:Rout - h], bot], axis=0)
    if f32_out:
        return y
    return y.astype(jnp.bfloat16)


def _pool2(x):
    B, H, W, C = x.shape
    y = x.astype(jnp.float32).reshape(B, H // 2, 2, W // 2, 2, C).mean(axis=(2, 4))
    return y.astype(x.dtype)


def _chain_body(*refs, n_layers, relus, gmeta, nb, TH, H, has_skip, skip_after,
                head_post):
    it = iter(refs)
    x_ref = next(it)
    skip_ref = next(it) if has_skip else None
    wbs = []
    for l in range(n_layers):
        wbs.append([(next(it), next(it)) for _ in gmeta[l]])
    out_refs = [next(it)]
    pads = [next(it) for _ in range(n_layers)]

    j = pl.program_id(0) % nb
    x = x_ref[0]
    h = n_layers
    for l in range(n_layers):
        h -= 1
        last = (l == n_layers - 1)
        groups = [(w, b, co, cl, m9)
                  for (w, b), (co, cl, m9) in zip(wbs[l], gmeta[l])]
        row_lo = h - j * TH
        row_hi = H + h - j * TH
        x = _conv_step(x, pads[l], groups, relus[l], row_lo, row_hi,
                       mask=h, f32_out=(last and head_post))
        if has_skip and l == skip_after:
            x = x + skip_ref[0]

    if head_post:
        k2 = _K2
        w1 = jax.nn.softmax(x[..., 0 * k2:1 * k2], axis=-1)
        w2 = jax.nn.softmax(x[..., 3 * k2:4 * k2], axis=-1)
        occ = jax.nn.sigmoid(x[..., 6 * k2:6 * k2 + 1])
        y = jnp.concatenate([w1, x[..., 1 * k2:3 * k2], w2,
                             x[..., 4 * k2:6 * k2], occ], axis=-1)
        out_refs[0][...] = y[None]
        return

    out_refs[0][...] = x[None]


def _band(x, halo, TH):
    B, H, W, C = x.shape
    nb = H // TH
    if halo == 0 and nb == 1:
        return x
    xp = jnp.pad(x, ((0, 0), (halo, halo), (0, 0), (0, 0)))
    bands = [xp[:, i * TH: i * TH + TH + 2 * halo] for i in range(nb)]
    return jnp.stack(bands, axis=1).reshape(B * nb, TH + 2 * halo, W, C)


def _conv_chain(x, layers, *, TH, skip=None, skip_after=0, head_post=False):
    B, H, W, Cin = x.shape
    L = len(layers)
    nb = H // TH
    R0 = TH + 2 * L
    xb = _band(x, L, TH)
    inputs = [xb]
    in_specs = [pl.BlockSpec((1, R0, W, Cin), lambda i: (i, 0, 0, 0))]
    has_skip = skip is not None
    if has_skip:
        hs = L - 1 - skip_after
        sb = _band(skip, hs, TH)
        inputs.append(sb)
        Rs = TH + 2 * hs
        Cs = skip.shape[-1]
        in_specs.append(pl.BlockSpec((1, Rs, W, Cs), lambda i: (i, 0, 0, 0)))
    relus = []
    gmeta = []
    scratch = []
    Rl = R0
    Cl = Cin
    for lay in layers:
        if len(lay) == 3:
            w, b, relu = lay
            grps = [(w, b, 0, Cl)]
        else:
            grps, relu = lay
        meta = []
        cout_sum = 0
        for (w, b, ci_off, ci_len) in grps:
            cout = w.shape[2]
            m9 = (9 * ci_len <= 1152
                  and Rl * W * 9 * ci_len * 2 <= 7 * 1024 * 1024)
            if m9:
                inputs.append(w.reshape(9 * (w.shape[1] // 3), cout))
                in_specs.append(pl.BlockSpec((9 * ci_len, cout),
                                             lambda i: (0, 0)))
            else:
                inputs.append(w)
                in_specs.append(pl.BlockSpec((3, w.shape[1], cout),
                                             lambda i: (0, 0, 0)))
            inputs.append(b)
            in_specs.append(pl.BlockSpec((1, cout), lambda i: (0, 0)))
            meta.append((ci_off, ci_len, m9))
            cout_sum += cout
        relus.append(relu)
        gmeta.append(tuple(meta))
        scratch.append(pltpu.VMEM((Rl, W + 2, Cl), jnp.bfloat16))
        Rl -= 2
        Cl = cout_sum
    Cfin = Cl
    odt = jnp.float32 if head_post else jnp.bfloat16
    out_shapes = [jax.ShapeDtypeStruct((B, H, W, Cfin), odt)]
    out_specs = [pl.BlockSpec((1, TH, W, Cfin),
                              lambda i, nb=nb: (i // nb, i % nb, 0, 0))]

    body = functools.partial(
        _chain_body, n_layers=L, relus=tuple(relus), gmeta=tuple(gmeta),
        nb=nb, TH=TH, H=H,
        has_skip=has_skip, skip_after=skip_after, head_post=head_post)
    res = pl.pallas_call(
        body,
        out_shape=tuple(out_shapes),
        grid_spec=pltpu.PrefetchScalarGridSpec(
            num_scalar_prefetch=0,
            grid=(B * nb,),
            in_specs=in_specs,
            out_specs=tuple(out_specs),
            scratch_shapes=scratch),
        compiler_params=pltpu.CompilerParams(
            dimension_semantics=("parallel",),
            vmem_limit_bytes=_VMEM_LIMIT),
    )(*inputs)
    return tuple(res)


_HL1_CO = (51, 50, 51, 50, 50, 51, 52)
_HL2_CO = (49, 48, 50, 49, 49, 50, 51)
_HL3_CO = (21, 20, 20, 20, 20, 20, 48)


def _l1_pad(w, b):
    w4 = w.reshape(3, 3, 51, 355)
    ws, bs = [], []
    off = 0
    for co in _HL1_CO:
        ws.append(jnp.pad(w4[..., off:off + co],
                          ((0, 0), (0, 0), (0, 0), (0, 64 - co))))
        bs.append(jnp.pad(b[:, off:off + co], ((0, 0), (0, 64 - co))))
        off += co
    return jnp.concatenate(ws, -1).reshape(3, 153, 448), jnp.concatenate(bs, -1)


def _blockdiag_split(w, b, cins, couts, cin_pads, cout_pads):
    tot_ci, tot_co = sum(cins), sum(couts)
    w4 = w.reshape(3, 3, tot_ci, tot_co)
    out = []
    ci_off = co_off = 0
    for ci, co, cip, cop in zip(cins, couts, cin_pads, cout_pads):
        blk = w4[:, :, ci_off:ci_off + ci, co_off:co_off + co]
        blk = jnp.pad(blk, ((0, 0), (0, 0), (0, cip - ci), (0, cop - co)))
        bg = jnp.pad(b[:, co_off:co_off + co], ((0, 0), (0, cop - co)))
        out.append((blk.reshape(3, 3 * cip, cop), bg))
        ci_off += ci
        co_off += co
    return out


def _l4_blocks(w, b):
    w4 = w.reshape(3, 3, 169, 151)
    ci_real = [0, 21, 41, 61, 81, 101, 121]
    wA = jnp.zeros((3, 3, 160, 125), w.dtype)
    for g in range(5):
        blk = w4[:, :, ci_real[g]:ci_real[g] + _HL3_CO[g], 25 * g:25 * (g + 1)]
        wA = wA.at[:, :, 32 * g:32 * g + _HL3_CO[g], 25 * g:25 * (g + 1)].set(blk)
    wB = jnp.zeros((3, 3, 96, 26), w.dtype)
    wB = wB.at[:, :, 0:20, 0:25].set(w4[:, :, 101:121, 125:150])
    wB = wB.at[:, :, 32:80, 25:26].set(w4[:, :, 121:169, 150:151])
    return ((wA.reshape(3, 480, 125), b[:, 0:125]),
            (wB.reshape(3, 288, 26), b[:, 125:151]))


@functools.lru_cache(maxsize=None)
def _up_mat(n_in):
    import numpy as np
    n_out = 2 * n_in
    src = np.arange(n_out) * ((n_in - 1) / (n_out - 1))
    i0 = np.clip(np.floor(src).astype(np.int64), 0, n_in - 2)
    f = (src - i0).astype(np.float32)
    M = np.zeros((n_out, n_in), np.float32)
    M[np.arange(n_out), i0] = 1.0 - f
    M[np.arange(n_out), i0 + 1] = f
    return M


def _up2(x):
    B, H, W, C = x.shape
    Mh = jnp.asarray(_up_mat(H), jnp.bfloat16)
    Mw = jnp.asarray(_up_mat(W), jnp.bfloat16)
    y = jnp.einsum('oh,bhwc->bowc', Mh, x,
                   preferred_element_type=jnp.float32).astype(jnp.bfloat16)
    return jnp.einsum('ow,bhwc->bhoc', Mw, y,
                      preferred_element_type=jnp.float32).astype(jnp.bfloat16)


def _adacof_pair(p0x, p2x, W1, A1, B1, W2, A2, B2):
    Bn, H, Wd, nt = W1.shape
    Hp = p0x.shape[1] - 2
    Wp = p0x.shape[2] - 2
    taps = jnp.arange(nt)
    ky = taps // _KS
    kx = taps % _KS
    ys = jnp.arange(H)[None, :, None, None]
    xs = jnp.arange(Wd)[None, None, :, None]

    def mk(Al, Be):
        a0 = jnp.floor(Al)
        b0 = jnp.floor(Be)
        fa = Al - a0
        fb = Be - b0
        iy0 = ys + ky[None, None, None, :] + a0.astype(jnp.int32)
        ix0 = xs + kx[None, None, None, :] + b0.astype(jnp.int32)
        sy = jnp.clip(iy0, -1, Hp - 1) + 1
        sx = jnp.clip(ix0, -1, Wp - 1) + 1
        return jnp.stack([sy, sx], axis=-1), fa, fb

    i1, fa1, fb1 = mk(A1, B1)
    i2, fa2, fb2 = mk(A2, B2)
    xg = jnp.concatenate([p0x, p2x], axis=0).astype(jnp.bfloat16)
    idx = jnp.concatenate([i1, i2], axis=0).reshape(2 * Bn, -1, 2)
    dnums = jax.lax.GatherDimensionNumbers(
        offset_dims=(1, 2, 3), collapsed_slice_dims=(),
        start_index_map=(0, 1))

    def g(img, ind):
        return jax.lax.gather(img, ind, dnums, (2, 2, 3),
                              mode=jax.lax.GatherScatterMode.PROMISE_IN_BOUNDS)

    patches = jax.vmap(g)(xg, idx).reshape(2 * Bn, H, Wd, nt, 2, 2, 3)
    v00 = patches[..., 0, 0, :]
    v01 = patches[..., 0, 1, :]
    v10 = patches[..., 1, 0, :]
    v11 = patches[..., 1, 1, :]
    fa = jnp.concatenate([fa1, fa2], axis=0)
    fb = jnp.concatenate([fb1, fb2], axis=0)
    w00 = ((1 - fa) * (1 - fb)).astype(jnp.bfloat16)[..., None]
    w01 = ((1 - fa) * fb).astype(jnp.bfloat16)[..., None]
    w10 = (fa * (1 - fb)).astype(jnp.bfloat16)[..., None]
    w11 = (fa * fb).astype(jnp.bfloat16)[..., None]
    samp = v00 * w00 + v10 * w10 + v01 * w01 + v11 * w11
    Wt = jnp.concatenate([W1, W2], axis=0).astype(jnp.bfloat16)
    t = jnp.einsum('bhwk,bhwkc->bhwc', Wt, samp,
                   preferred_element_type=jnp.float32)
    return t[:Bn], t[Bn:]


def kernel(frame0, frame2, conv1_0w, conv1_0b, conv1_1w, conv1_1b, conv1_2w, conv1_2b, conv2_0w, conv2_0b, conv2_1w, conv2_1b, conv2_2w, conv2_2b, conv3_0w, conv3_0b, conv3_1w, conv3_1b, conv3_2w, conv3_2b, conv4_0w, conv4_0b, conv4_1w, conv4_1b, conv4_2w, conv4_2b, conv5_0w, conv5_0b, conv5_1w, conv5_1b, conv5_2w, conv5_2b, deconv5_0w, deconv5_0b, deconv5_1w, deconv5_1b, deconv5_2w, deconv5_2b, up5_0w, up5_0b, deconv4_0w, deconv4_0b, deconv4_1w, deconv4_1b, deconv4_2w, deconv4_2b, up4_0w, up4_0b, deconv3_0w, deconv3_0b, deconv3_1w, deconv3_1b, deconv3_2w, deconv3_2b, up3_0w, up3_0b, deconv2_0w, deconv2_0b, deconv2_1w, deconv2_1b, deconv2_2w, deconv2_2b, up2_0w, up2_0b, head_l1_w, head_l1_b, head_l2_w, head_l2_b, head_l3_w, head_l3_b, head_l4_w, head_l4_b):
    f0 = jnp.transpose(frame0, (0, 2, 3, 1)).astype(jnp.float32)
    f2 = jnp.transpose(frame2, (0, 2, 3, 1)).astype(jnp.float32)
    H, W = f0.shape[1], f0.shape[2]
    n0 = (f0 - _MEANS).astype(jnp.bfloat16)
    n2 = (f2 - _MEANS).astype(jnp.bfloat16)
    x = jnp.concatenate([n0, n2], axis=-1)

    (c1,) = _conv_chain(x, [(conv1_0w, conv1_0b, True),
                            (conv1_1w, conv1_1b, True),
                            (conv1_2w, conv1_2b, True)], TH=32)
    p1 = _pool2(c1)
    (c2,) = _conv_chain(p1, [(conv2_0w, conv2_0b, True),
                             (conv2_1w, conv2_1b, True),
                             (conv2_2w, conv2_2b, True)], TH=32)
    p2 = _pool2(c2)
    (c3,) = _conv_chain(p2, [(conv3_0w, conv3_0b, True),
                             (conv3_1w, conv3_1b, True),
                             (conv3_2w, conv3_2b, True)], TH=64)
    p3 = _pool2(c3)
    (c4,) = _conv_chain(p3, [(conv4_0w, conv4_0b, True),
                             (conv4_1w, conv4_1b, True),
                             (conv4_2w, conv4_2b, True)], TH=32)
    p4 = _pool2(c4)
    (c5,) = _conv_chain(p4, [(conv5_0w, conv5_0b, True),
                             (conv5_1w, conv5_1b, True),
                             (conv5_2w, conv5_2b, True)], TH=16)
    p5 = _pool2(c5)
    (d5,) = _conv_chain(p5, [(deconv5_0w, deconv5_0b, True),
                             (deconv5_1w, deconv5_1b, True),
                             (deconv5_2w, deconv5_2b, True)], TH=8)

    (d4,) = _conv_chain(_up2(d5), [(up5_0w, up5_0b, True),
                                   (deconv4_0w, deconv4_0b, True),
                                   (deconv4_1w, deconv4_1b, True),
                                   (deconv4_2w, deconv4_2b, True)], TH=16,
                        skip=c5, skip_after=0)
    (d3,) = _conv_chain(_up2(d4), [(up4_0w, up4_0b, True),
                                   (deconv3_0w, deconv3_0b, True),
                                   (deconv3_1w, deconv3_1b, True),
                                   (deconv3_2w, deconv3_2b, True)], TH=32,
                        skip=c4, skip_after=0)
    (d2,) = _conv_chain(_up2(d3), [(up3_0w, up3_0b, True),
                                   (deconv2_0w, deconv2_0b, True),
                                   (deconv2_1w, deconv2_1b, True),
                                   (deconv2_2w, deconv2_2b, True)], TH=64,
                        skip=c3, skip_after=0)
    l1w, l1b = _l1_pad(head_l1_w, head_l1_b)
    l2g = _blockdiag_split(head_l2_w, head_l2_b, _HL1_CO, _HL2_CO,
                           [64] * 7, [64] * 7)
    l3g = _blockdiag_split(head_l3_w, head_l3_b, _HL2_CO, _HL3_CO,
                           [64] * 7, [32] * 6 + [64])
    (h3,) = _conv_chain(_up2(d2), [(up2_0w, up2_0b, True),
                                   (l1w, l1b, True),
                                   ([(w, b, 64 * g, 64)
                                     for g, (w, b) in enumerate(l2g)], True),
                                   ([(w, b, 64 * g, 64)
                                     for g, (w, b) in enumerate(l3g)], True)],
                        TH=32, skip=c2, skip_after=0)
    wA, wB = _l4_blocks(head_l4_w, head_l4_b)
    (hh,) = _conv_chain(_up2(h3), [([(wA[0], wA[1], 0, 160),
                                     (wB[0], wB[1], 160, 96)], False)], TH=32,
                        head_post=True)

    W1 = hh[..., 0 * _K2:1 * _K2]
    A1 = hh[..., 1 * _K2:2 * _K2]
    B1 = hh[..., 2 * _K2:3 * _K2]
    W2 = hh[..., 3 * _K2:4 * _K2]
    A2 = hh[..., 4 * _K2:5 * _K2]
    B2 = hh[..., 5 * _K2:6 * _K2]
    Occ = hh[..., 6 * _K2:6 * _K2 + 1]

    kp = (_KS - 1) // 2 + 1
    cfg = ((0, 0), (kp, kp), (kp, kp), (0, 0))
    p0 = jnp.pad(f0, cfg, mode='edge')
    p2 = jnp.pad(f2, cfg, mode='edge')
    t1, t2 = _adacof_pair(p0, p2, W1, A1, B1, W2, A2, B2)
    frame1 = Occ * t1 + (1.0 - Occ) * t2
    return jnp.transpose(frame1, (0, 3, 1, 2))
